# baseline (speedup 1.0000x reference)
import functools

import jax
import jax.numpy as jnp
from jax import lax
from jax.experimental import pallas as pl
from jax.experimental.pallas import tpu as pltpu
from jax.experimental.pallas import tpu_sc as plsc

_NC = 2
_NS = 16
_NW = _NC * _NS
_BUF_ROWS = 32


def _make_sc_copy(rows, d, dtype):
    rows_per_w = rows // _NW
    n_iter = rows_per_w // _BUF_ROWS
    mesh = plsc.VectorSubcoreMesh(core_axis_name="c", subcore_axis_name="s")

    @functools.partial(
        pl.kernel,
        mesh=mesh,
        out_type=jax.ShapeDtypeStruct((rows, d), dtype),
        scratch_types=[
            pltpu.VMEM((_BUF_ROWS, d), dtype),
            pltpu.SemaphoreType.DMA,
        ],
    )
    def k(x_hbm, out_hbm, buf, sem):
        wid = lax.axis_index("s") * _NC + lax.axis_index("c")
        base = wid * rows_per_w

        @pl.loop(0, n_iter)
        def _(i):
            off = base + i * _BUF_ROWS
            pltpu.sync_copy(x_hbm.at[pl.ds(off, _BUF_ROWS), :], buf)
            pltpu.sync_copy(buf, out_hbm.at[pl.ds(off, _BUF_ROWS), :])

    return k


def kernel(x):
    b, s, d = x.shape
    rows = b * s
    x2 = x.reshape(rows, d)
    out = _make_sc_copy(rows, d, x.dtype)(x2)
    return out.reshape(b, s, d)


# speedup vs baseline: 1.3183x; 1.3183x over previous
import jax
import jax.numpy as jnp
from jax.experimental import pallas as pl
from jax.experimental.pallas import tpu as pltpu

_ROWS = 32768
_D = 2048
_CHUNK = 2048


def _tc_body(x_hbm, out_hbm):
    n = _ROWS // _CHUNK

    def scoped(buf0, buf1, rs0, rs1, ws0, ws1):
        bufs = (buf0, buf1)
        rsems = (rs0, rs1)
        wsems = (ws0, ws1)
        reads = [
            pltpu.make_async_copy(
                x_hbm.at[pl.ds(i * _CHUNK, _CHUNK), :],
                bufs[i % 2],
                rsems[i % 2],
            )
            for i in range(n)
        ]
        writes = [
            pltpu.make_async_copy(
                bufs[i % 2],
                out_hbm.at[pl.ds(i * _CHUNK, _CHUNK), :],
                wsems[i % 2],
            )
            for i in range(n)
        ]
        reads[0].start()
        for i in range(n):
            if i + 1 < n:
                if i >= 1:
                    writes[i - 1].wait()
                reads[i + 1].start()
            reads[i].wait()
            writes[i].start()
        if n >= 2:
            writes[n - 2].wait()
        writes[n - 1].wait()

    pl.run_scoped(
        scoped,
        pltpu.VMEM((_CHUNK, _D), jnp.float32),
        pltpu.VMEM((_CHUNK, _D), jnp.float32),
        pltpu.SemaphoreType.DMA,
        pltpu.SemaphoreType.DMA,
        pltpu.SemaphoreType.DMA,
        pltpu.SemaphoreType.DMA,
    )


def _make_copy():
    tc_mesh = pltpu.create_tensorcore_mesh("tc")
    return pl.kernel(
        _tc_body,
        mesh=tc_mesh,
        out_type=jax.ShapeDtypeStruct((_ROWS, _D), jnp.float32),
    )


def kernel(x):
    b, s, d = x.shape
    x2 = x.reshape(b * s, d)
    out = _make_copy()(x2)
    return out.reshape(b, s, d)


# speedup vs baseline: 1.3324x; 1.0107x over previous
import jax
import jax.numpy as jnp
from jax.experimental import pallas as pl
from jax.experimental.pallas import tpu as pltpu


def _copy_block(x_ref, o_ref):
    o_ref[...] = x_ref[...]


def kernel(x):
    b, s, d = x.shape
    x2 = x.reshape(b * s, d)
    rows = b * s
    block_rows = 1984
    grid = pl.cdiv(rows, block_rows)
    out = pl.pallas_call(
        _copy_block,
        grid=(grid,),
        in_specs=[pl.BlockSpec((block_rows, d), lambda i: (i, 0))],
        out_specs=pl.BlockSpec((block_rows, d), lambda i: (i, 0)),
        out_shape=jax.ShapeDtypeStruct((rows, d), x.dtype),
        compiler_params=pltpu.CompilerParams(
            dimension_semantics=("parallel",),
            vmem_limit_bytes=67108864,
        ),
    )(x2)
    return out.reshape(b, s, d)


# speedup vs baseline: 1.3328x; 1.0003x over previous
import jax
import jax.numpy as jnp
from jax.experimental import pallas as pl
from jax.experimental.pallas import tpu as pltpu


def _copy_block(x_ref, o_ref):
    o_ref[...] = x_ref[...]


def kernel(x):
    b, s, d = x.shape
    x2 = x.reshape(b * s, d)
    rows = b * s
    block_rows = 2016
    grid = pl.cdiv(rows, block_rows)
    out = pl.pallas_call(
        _copy_block,
        grid=(grid,),
        in_specs=[pl.BlockSpec((block_rows, d), lambda i: (i, 0))],
        out_specs=pl.BlockSpec((block_rows, d), lambda i: (i, 0)),
        out_shape=jax.ShapeDtypeStruct((rows, d), x.dtype),
        compiler_params=pltpu.CompilerParams(
            dimension_semantics=("parallel",),
            vmem_limit_bytes=67108864,
        ),
    )(x2)
    return out.reshape(b, s, d)
